# Initial kernel scaffold; baseline (speedup 1.0000x reference)
import jax
import jax.numpy as jnp
from jax.experimental import pallas as pl


def kernel(scores, labels, idx_i, idx_j):
    raise NotImplementedError("write your pallas kernel here")


# speedup vs baseline: 109.3894x; 109.3894x over previous
import functools

import jax
import jax.numpy as jnp
from jax import lax
from jax.experimental import pallas as pl
from jax.experimental.pallas import tpu as pltpu
from jax.experimental.pallas import tpu_sc as plsc

_SIGMA = 1.0
_EPS = 1e-7
_L = 16
_NC = 2
_NS = 16
_NW = _NC * _NS
_C = 2560


def _logf(x):
    bits = lax.bitcast_convert_type(x, jnp.int32)
    e = ((bits >> 23) & 0xFF) - 126
    m = lax.bitcast_convert_type((bits & 0x007FFFFF) | 0x3F000000, jnp.float32)
    lt = m < 0.7071067811865476
    e = e - lt.astype(jnp.int32)
    f = jnp.where(lt, m + m, m) - 1.0
    z = f * f
    p = jnp.full_like(f, 7.0376836292e-2)
    for c in (-1.1514610310e-1, 1.1676998740e-1, -1.2420140846e-1,
              1.4249322787e-1, -1.6668057665e-1, 2.0000714765e-1,
              -2.4999993993e-1, 3.3333331174e-1):
        p = p * f + c
    ef = e.astype(jnp.float32)
    y = p * f * z + ef * (-2.12194440e-4) - 0.5 * z
    return f + y + ef * 0.693359375


@functools.lru_cache(maxsize=None)
def _build(n_chunks_per_worker):
    g_count = n_chunks_per_worker

    def body(table, idxi, idxj, out,
             ii0, ij0, ri0, rj0, ii1, ij1, ri1, rj1, accvm, sem0, sem1):
        cid = lax.axis_index("c")
        sid = lax.axis_index("s")
        wid = sid * _NC + cid
        base = wid * g_count
        ibufs = (ii0, ii1)
        jbufs = (ij0, ij1)
        ribufs = (ri0, ri1)
        rjbufs = (rj0, rj1)
        sems = (sem0, sem1)

        def start(g, b):
            off = (base + g) * _C
            pltpu.sync_copy(idxi.at[pl.ds(off, _C)], ibufs[b])
            pltpu.sync_copy(idxj.at[pl.ds(off, _C)], jbufs[b])
            pltpu.async_copy(table.at[ibufs[b]], ribufs[b], sems[b])
            pltpu.async_copy(table.at[jbufs[b]], rjbufs[b], sems[b])

        def wait(b):
            pltpu.make_async_copy(table.at[ibufs[b]], ribufs[b], sems[b]).wait()
            pltpu.make_async_copy(table.at[jbufs[b]], rjbufs[b], sems[b]).wait()

        iota = lax.iota(jnp.int32, _L)
        zeros_i = jnp.zeros((_L,), jnp.int32)
        ones_i = jnp.ones((_L,), jnp.int32)

        def chunk_sum(b, acc, cnt):
            ib, jb, rib, rjb = ibufs[b], jbufs[b], ribufs[b], rjbufs[b]

            def kbody(k, carry):
                a, c2 = carry
                sl = pl.ds(k * _L, _L)
                iv = ib[sl]
                jv = jb[sl]
                rid = iota + k * _L
                s_i = plsc.load_gather(rib, [rid, zeros_i])
                y_i = plsc.load_gather(rib, [rid, ones_i])
                s_j = plsc.load_gather(rjb, [rid, zeros_i])
                y_j = plsc.load_gather(rjb, [rid, ones_i])
                d = _SIGMA * (s_i - s_j)
                pred = 1.0 / (1.0 + jnp.exp(-d))
                lp = _logf(pred + _EPS)
                lq = _logf((1.0 - pred) + _EPS)
                t = (jnp.sign(y_i - y_j) + 1.0) * 0.5
                loss = -(t * lp + (1.0 - t) * lq)
                v = iv != jv
                a = a + jnp.where(v, loss, 0.0)
                c2 = c2 + jnp.where(v, 1.0, 0.0)
                return a, c2

            return lax.fori_loop(0, _C // _L, kbody, (acc, cnt))

        start(0, 0)
        zf = jnp.zeros((_L,), jnp.float32)

        @pl.loop(0, g_count, step=2, init_carry=(zf, zf))
        def run(t, carry):
            acc, cnt = carry
            for b in (0, 1):
                g = t + b

                @pl.when(g + 1 < g_count)
                def _():
                    start(g + 1, 1 - b)

                wait(b)
                acc, cnt = chunk_sum(b, acc, cnt)
            return acc, cnt

        acc, cnt = run
        accvm[0, :] = acc
        accvm[1, :] = cnt
        pltpu.sync_copy(accvm, out.at[wid])

    mesh = plsc.VectorSubcoreMesh(core_axis_name="c", subcore_axis_name="s")
    return pl.kernel(
        body,
        out_type=jax.ShapeDtypeStruct((_NW, 2, _L), jnp.float32),
        mesh=mesh,
        compiler_params=pltpu.CompilerParams(needs_layout_passes=False,
                                             use_tc_tiling_on_sc=False),
        scratch_types=[
            pltpu.VMEM((_C,), jnp.int32),
            pltpu.VMEM((_C,), jnp.int32),
            pltpu.VMEM((_C, 2), jnp.float32),
            pltpu.VMEM((_C, 2), jnp.float32),
            pltpu.VMEM((_C,), jnp.int32),
            pltpu.VMEM((_C,), jnp.int32),
            pltpu.VMEM((_C, 2), jnp.float32),
            pltpu.VMEM((_C, 2), jnp.float32),
            pltpu.VMEM((2, _L), jnp.float32),
            pltpu.SemaphoreType.DMA,
            pltpu.SemaphoreType.DMA,
        ],
    )


def kernel(scores, labels, idx_i, idx_j):
    n_pairs = idx_i.shape[0]
    per_round = _NW * _C
    g_count = -(-n_pairs // per_round)
    if g_count % 2:
        g_count += 1
    padded = g_count * per_round
    pad = padded - n_pairs

    table = jnp.stack([scores.astype(jnp.float32),
                       labels.astype(jnp.float32)], axis=1)
    ii = jnp.concatenate([idx_i.astype(jnp.int32),
                          jnp.zeros((pad,), jnp.int32)])
    jj = jnp.concatenate([idx_j.astype(jnp.int32),
                          jnp.zeros((pad,), jnp.int32)])

    parts = _build(g_count)(table, ii, jj)
    return jnp.sum(parts[:, 0, :]) / jnp.sum(parts[:, 1, :])


# speedup vs baseline: 272.1236x; 2.4877x over previous
import functools

import jax
import jax.numpy as jnp
from jax import lax
from jax.experimental import pallas as pl
from jax.experimental.pallas import tpu as pltpu
from jax.experimental.pallas import tpu_sc as plsc

_SIGMA = 1.0
_EPS = 1e-7
_L = 16
_NC = 2
_NS = 16
_NW = _NC * _NS
_C = 2000


def _logf(x):
    bits = lax.bitcast_convert_type(x, jnp.int32)
    e = ((bits >> 23) & 0xFF) - 126
    m = lax.bitcast_convert_type((bits & 0x007FFFFF) | 0x3F000000, jnp.float32)
    lt = m < 0.7071067811865476
    e = e - lt.astype(jnp.int32)
    f = jnp.where(lt, m + m, m) - 1.0
    z = f * f
    p = jnp.full_like(f, 7.0376836292e-2)
    for c in (-1.1514610310e-1, 1.1676998740e-1, -1.2420140846e-1,
              1.4249322787e-1, -1.6668057665e-1, 2.0000714765e-1,
              -2.4999993993e-1, 3.3333331174e-1):
        p = p * f + c
    ef = e.astype(jnp.float32)
    y = p * f * z + ef * (-2.12194440e-4) - 0.5 * z
    return f + y + ef * 0.693359375


@functools.lru_cache(maxsize=None)
def _build(n_chunks):
    base_chunks = n_chunks // _NW
    extra = n_chunks % _NW
    g_loop = base_chunks + (1 if extra else 0)
    if g_loop % 2:
        g_loop += 1

    def body(scores, labels, idxi, idxj, out,
             ii0, ij0, si0, yi0, sj0, yj0,
             ii1, ij1, si1, yi1, sj1, yj1, accvm, sem0, sem1):
        cid = lax.axis_index("c")
        sid = lax.axis_index("s")
        wid = sid * _NC + cid
        start_chunk = wid * base_chunks + jnp.minimum(wid, extra)
        my_chunks = base_chunks + jnp.where(wid < extra, 1, 0)
        ibufs = (ii0, ii1)
        jbufs = (ij0, ij1)
        sibufs = (si0, si1)
        yibufs = (yi0, yi1)
        sjbufs = (sj0, sj1)
        yjbufs = (yj0, yj1)
        sems = (sem0, sem1)

        def start(g, b):
            off = jnp.minimum(start_chunk + g, n_chunks - 1) * _C
            pltpu.sync_copy(idxi.at[pl.ds(off, _C)], ibufs[b])
            pltpu.sync_copy(idxj.at[pl.ds(off, _C)], jbufs[b])
            pltpu.async_copy(scores.at[ibufs[b]], sibufs[b], sems[b])
            pltpu.async_copy(labels.at[ibufs[b]], yibufs[b], sems[b])
            pltpu.async_copy(scores.at[jbufs[b]], sjbufs[b], sems[b])
            pltpu.async_copy(labels.at[jbufs[b]], yjbufs[b], sems[b])

        def wait(b):
            pltpu.make_async_copy(scores.at[ibufs[b]], sibufs[b], sems[b]).wait()
            pltpu.make_async_copy(labels.at[ibufs[b]], yibufs[b], sems[b]).wait()
            pltpu.make_async_copy(scores.at[jbufs[b]], sjbufs[b], sems[b]).wait()
            pltpu.make_async_copy(labels.at[jbufs[b]], yjbufs[b], sems[b]).wait()

        def chunk_sum(b):
            ib, jb = ibufs[b], jbufs[b]
            sib, yib, sjb, yjb = sibufs[b], yibufs[b], sjbufs[b], yjbufs[b]

            def kbody(k, carry):
                a, c2 = carry
                sl = pl.ds(k * _L, _L)
                iv = ib[sl]
                jv = jb[sl]
                s_i = sib[sl]
                y_i = yib[sl]
                s_j = sjb[sl]
                y_j = yjb[sl]
                d = _SIGMA * (s_i - s_j)
                pred = 1.0 / (1.0 + jnp.exp(-d))
                lp = _logf(pred + _EPS)
                lq = _logf((1.0 - pred) + _EPS)
                t = (jnp.sign(y_i - y_j) + 1.0) * 0.5
                loss = -(t * lp + (1.0 - t) * lq)
                v = iv != jv
                a = a + jnp.where(v, loss, 0.0)
                c2 = c2 + jnp.where(v, 1.0, 0.0)
                return a, c2

            zf = jnp.zeros((_L,), jnp.float32)
            return lax.fori_loop(0, _C // _L, kbody, (zf, zf))

        start(0, 0)
        zf = jnp.zeros((_L,), jnp.float32)

        @pl.loop(0, g_loop, step=2, init_carry=(zf, zf))
        def run(t, carry):
            acc, cnt = carry
            for b in (0, 1):
                g = t + b
                start(g + 1, 1 - b)
                wait(b)
                lw = jnp.where(g < my_chunks, 1.0, 0.0).astype(jnp.float32)
                ca, cc = chunk_sum(b)
                acc = acc + lw * ca
                cnt = cnt + lw * cc
            return acc, cnt

        acc, cnt = run
        wait(g_loop % 2)
        accvm[0, :] = acc
        accvm[1, :] = cnt
        pltpu.sync_copy(accvm, out.at[wid])

    mesh = plsc.VectorSubcoreMesh(core_axis_name="c", subcore_axis_name="s",
                                  num_cores=_NC, num_subcores=_NS)
    idxbuf = pltpu.VMEM((_C,), jnp.int32)
    valbuf = pltpu.VMEM((_C,), jnp.float32)
    return pl.kernel(
        body,
        out_type=jax.ShapeDtypeStruct((_NW, 2, _L), jnp.float32),
        mesh=mesh,
        compiler_params=pltpu.CompilerParams(needs_layout_passes=False,
                                             use_tc_tiling_on_sc=False),
        scratch_types=[
            idxbuf, idxbuf, valbuf, valbuf, valbuf, valbuf,
            idxbuf, idxbuf, valbuf, valbuf, valbuf, valbuf,
            pltpu.VMEM((2, _L), jnp.float32),
            pltpu.SemaphoreType.DMA,
            pltpu.SemaphoreType.DMA,
        ],
    )


def kernel(scores, labels, idx_i, idx_j):
    n_pairs = idx_i.shape[0]
    if n_pairs % _C:
        raise ValueError("n_pairs must be a multiple of the chunk size")
    n_chunks = n_pairs // _C

    parts = _build(n_chunks)(scores.astype(jnp.float32),
                             labels.astype(jnp.float32),
                             idx_i.astype(jnp.int32),
                             idx_j.astype(jnp.int32))
    return jnp.sum(parts[:, 0, :]) / jnp.sum(parts[:, 1, :])
